# baseline (speedup 1.0000x reference)
import functools

import jax
import jax.numpy as jnp
import numpy as np
from jax import lax
from jax.experimental import pallas as pl
from jax.experimental.pallas import tpu as pltpu
from jax.experimental.pallas import tpu_sc as plsc

_N = 524288
_NW = 16
_CHUNK = _N // _NW
_VPC = _CHUNK // 16
_UNROLL = 16
_ROUNDS = 32
_MINI32 = np.int32(-2147483648)
_MAXI32 = np.int32(0x7FFFFFFF)
_ONEI32 = np.int32(1)


def _keyify(x, t):
    b = lax.bitcast_convert_type(x, jnp.int32)
    key = jnp.where(b >= 0, b, b ^ _MAXI32)
    return jnp.where(t > 0.0, _MINI32, key)


def _sc_select(x, t):
    mesh = plsc.VectorSubcoreMesh(
        core_axis_name="c", subcore_axis_name="s", num_cores=1
    )

    @functools.partial(
        pl.kernel,
        out_type=jax.ShapeDtypeStruct((16,), jnp.int32),
        mesh=mesh,
        compiler_params=pltpu.CompilerParams(needs_layout_passes=False),
        scratch_types=[
            pltpu.VMEM((_CHUNK,), jnp.float32),
            pltpu.VMEM((_CHUNK,), jnp.float32),
            pltpu.VMEM((_CHUNK,), jnp.int32),
            pltpu.VMEM((16,), jnp.int32),
            pltpu.VMEM((256,), jnp.int32),
            pltpu.VMEM_SHARED(((_ROUNDS + 1) * 256,), jnp.int32),
        ],
    )
    def sel(x_hbm, t_hbm, out_hbm, xv, tv, kv, stage, rd, shared):
        s = lax.axis_index("s")
        base = s * _CHUNK
        pltpu.sync_copy(x_hbm.at[pl.ds(base, _CHUNK)], xv)
        pltpu.sync_copy(t_hbm.at[pl.ds(base, _CHUNK)], tv)

        def merge(slot, part):
            stage[...] = part
            pltpu.sync_copy(stage, shared.at[pl.ds(slot * 256 + s * 16, 16)])
            plsc.subcore_barrier()
            pltpu.sync_copy(shared.at[pl.ds(slot * 256, 256)], rd)
            tot = rd[pl.ds(0, 16)]
            for j in range(1, 16):
                tot = tot + rd[pl.ds(j * 16, 16)]
            return tot

        def p0(i, pc):
            o = i * (16 * _UNROLL)
            for j in range(_UNROLL):
                xj = xv[pl.ds(o + j * 16, 16)]
                tj = tv[pl.ds(o + j * 16, 16)]
                kv[pl.ds(o + j * 16, 16)] = _keyify(xj, tj)
                pc = pc + plsc.all_reduce_population_count(tj > 0.0)
            return pc

        pos_part = lax.fori_loop(
            0, _VPC // _UNROLL, p0, jnp.zeros((16,), jnp.int32)
        )
        pos_num = merge(0, pos_part)
        ki = pos_num * 3

        lo = jnp.full((16,), _MINI32 + _ONEI32, jnp.int32)
        hi = jnp.full((16,), _MAXI32, jnp.int32)
        one = jnp.full((16,), _ONEI32, jnp.int32)
        for r in range(_ROUNDS):
            d = hi - lo
            mid = lo + lax.shift_right_logical(d, one) + (d & one)

            def cstep(i, accs, mid=mid):
                o = i * (16 * _UNROLL)
                accs = list(accs)
                for j in range(_UNROLL):
                    u = kv[pl.ds(o + j * 16, 16)]
                    accs[j % 4] = accs[j % 4] + plsc.all_reduce_population_count(
                        u >= mid)
                return tuple(accs)

            z = jnp.zeros((16,), jnp.int32)
            a0, a1, a2, a3 = lax.fori_loop(
                0, _VPC // _UNROLL, cstep, (z, z, z, z)
            )
            cnt = merge(r + 1, a0 + a1 + a2 + a3)
            ok = cnt >= ki
            lo = jnp.where(ok, mid, lo)
            hi = jnp.where(ok, hi, mid - one)

        @pl.when(s == 0)
        def _():
            stage[...] = lo
            pltpu.sync_copy(stage, out_hbm)

    return sel(x, t)


_ROWS = 512
_COLS = 1024


def _tc_body(x_ref, t_ref, v_ref, out_ref):
    x = x_ref[...]
    t = t_ref[...]
    v = v_ref[0]
    b = jax.lax.bitcast_convert_type(x, jnp.int32)
    key = jnp.where(b >= 0, b, b ^ _MAXI32)
    is_pos = t > 0.0
    key = jnp.where(is_pos, _MINI32, key)
    pos_num_f = jnp.sum(jnp.where(is_pos, 1.0, 0.0))
    k = (pos_num_f * 3.0).astype(jnp.int32)

    vb = jnp.where(v >= 0, v, v ^ _MAXI32)
    x_v = jax.lax.bitcast_convert_type(vb, jnp.float32)

    softplus = jnp.maximum(x, 0.0) + jnp.log1p(jnp.exp(-jnp.abs(x)))
    gt = key > v
    count_gt = jnp.sum(jnp.where(gt, 1, 0))
    sum_gt = jnp.sum(jnp.where(gt, softplus, 0.0))
    pos_sum = jnp.sum(jnp.where(is_pos, softplus - x * t, 0.0))
    sp_v = jnp.maximum(x_v, 0.0) + jnp.log1p(jnp.exp(-jnp.abs(x_v)))
    tie_sum = (k - count_gt).astype(jnp.float32) * sp_v
    total = pos_num_f + k.astype(jnp.float32)
    out_ref[0, 0] = (pos_sum + sum_gt + tie_sum) / total


def kernel(input, target):
    v = _sc_select(input, target)
    x2 = input.reshape(_ROWS, _COLS)
    t2 = target.reshape(_ROWS, _COLS)
    out = pl.pallas_call(
        _tc_body,
        out_shape=jax.ShapeDtypeStruct((1, 1), jnp.float32),
        in_specs=[
            pl.BlockSpec(memory_space=pltpu.VMEM),
            pl.BlockSpec(memory_space=pltpu.VMEM),
            pl.BlockSpec(memory_space=pltpu.SMEM),
        ],
        out_specs=pl.BlockSpec(memory_space=pltpu.SMEM),
    )(x2, t2, v)
    return out[0, 0]


# speedup vs baseline: 1.1058x; 1.1058x over previous
import functools

import jax
import jax.numpy as jnp
import numpy as np
from jax import lax
from jax.experimental import pallas as pl
from jax.experimental.pallas import tpu as pltpu
from jax.experimental.pallas import tpu_sc as plsc

_N = 524288
_NW = 16
_CHUNK = _N // _NW
_VPC = _CHUNK // 16
_UNROLL = 8
_ROUNDS = 32
_MINI32 = np.int32(-2147483648)
_MAXI32 = np.int32(0x7FFFFFFF)
_ONEI32 = np.int32(1)


def _fkey(f):
    return np.frombuffer(np.float32(f).tobytes(), np.int32)[0]


_KA = _fkey(0.90)
_KB = _fkey(1.03)
_KC = _fkey(1.12)


def _keyify(x, t):
    b = lax.bitcast_convert_type(x, jnp.int32)
    key = jnp.where(b >= 0, b, b ^ _MAXI32)
    return jnp.where(t > 0.0, _MINI32, key)


def _sc_select(x, t):
    mesh = plsc.VectorSubcoreMesh(
        core_axis_name="c", subcore_axis_name="s", num_cores=1
    )

    @functools.partial(
        pl.kernel,
        out_type=jax.ShapeDtypeStruct((16,), jnp.int32),
        mesh=mesh,
        compiler_params=pltpu.CompilerParams(needs_layout_passes=False),
        scratch_types=[
            pltpu.VMEM((_CHUNK,), jnp.float32),
            pltpu.VMEM((_CHUNK,), jnp.float32),
            pltpu.VMEM((_CHUNK,), jnp.int32),
            pltpu.VMEM((64,), jnp.int32),
            pltpu.VMEM((1024,), jnp.int32),
            pltpu.VMEM_SHARED(((_ROUNDS + 2) * 1024,), jnp.int32),
        ],
    )
    def sel(x_hbm, t_hbm, out_hbm, xv, tv, kv, stage, rd, shared):
        s = lax.axis_index("s")
        base = s * _CHUNK
        pltpu.sync_copy(x_hbm.at[pl.ds(base, _CHUNK)], xv)
        pltpu.sync_copy(t_hbm.at[pl.ds(base, _CHUNK)], tv)

        def merge(slot, rows):
            for ri, row in enumerate(rows):
                stage[pl.ds(ri * 16, 16)] = row
            pltpu.sync_copy(stage, shared.at[pl.ds(slot * 1024 + s * 64, 64)])
            plsc.subcore_barrier()
            pltpu.sync_copy(shared.at[pl.ds(slot * 1024, 1024)], rd)
            out = []
            for ri in range(len(rows)):
                tot = rd[pl.ds(ri * 16, 16)]
                for j in range(1, 16):
                    tot = tot + rd[pl.ds(j * 64 + ri * 16, 16)]
                out.append(tot)
            return out

        def p0(i, pc):
            o = i * (16 * _UNROLL)
            for j in range(_UNROLL):
                xj = xv[pl.ds(o + j * 16, 16)]
                tj = tv[pl.ds(o + j * 16, 16)]
                kv[pl.ds(o + j * 16, 16)] = _keyify(xj, tj)
                pc = pc + plsc.all_reduce_population_count(tj > 0.0)
            return pc

        pos_part = lax.fori_loop(
            0, _VPC // _UNROLL, p0, jnp.zeros((16,), jnp.int32)
        )

        def bstep(i, accs):
            aa, ab, ac = accs
            o = i * (16 * _UNROLL)
            for j in range(_UNROLL):
                u = kv[pl.ds(o + j * 16, 16)]
                aa = aa + plsc.all_reduce_population_count(u >= _KA)
                ab = ab + plsc.all_reduce_population_count(u >= _KB)
                ac = ac + plsc.all_reduce_population_count(u >= _KC)
            return (aa, ab, ac)

        z = jnp.zeros((16,), jnp.int32)
        ba, bb, bc = lax.fori_loop(0, _VPC // _UNROLL, bstep, (z, z, z))
        g = merge(0, [ba, bb, bc, pos_part])
        pos_num = g[3][0]
        ki = pos_num * np.int32(3)
        ca, cb, cc = g[0][0], g[1][0], g[2][0]

        ok_in = (ca >= ki) & (cc < ki)
        use_b = cb >= ki
        lo = jnp.where(ok_in, jnp.where(use_b, _KB, _KA),
                       jnp.full((), _MINI32 + _ONEI32, jnp.int32))
        hi = jnp.where(ok_in,
                       jnp.where(use_b, _KC - _ONEI32, _KB - _ONEI32),
                       jnp.full((), _MAXI32, jnp.int32))

        for r in range(_ROUNDS):
            d = hi - lo
            mid = lo + lax.shift_right_logical(d, _ONEI32) + (d & _ONEI32)
            conv = hi > lo

            def cpass(mid=mid):
                def cstep(i, accs, mid=mid):
                    a0, a1 = accs
                    o = i * (16 * _UNROLL)
                    for j in range(_UNROLL):
                        u = kv[pl.ds(o + j * 16, 16)]
                        pc = plsc.all_reduce_population_count(u >= mid)
                        if j % 2 == 0:
                            a0 = a0 + pc
                        else:
                            a1 = a1 + pc
                    return (a0, a1)

                a0, a1 = lax.fori_loop(
                    0, _VPC // _UNROLL, cstep,
                    (jnp.zeros((16,), jnp.int32), jnp.zeros((16,), jnp.int32)))
                return a0 + a1

            part = lax.cond(conv, cpass, lambda: jnp.zeros((16,), jnp.int32))
            cnt = merge(r + 1, [part])[0][0]
            ok = conv & (cnt >= ki)
            lo = jnp.where(ok, mid, lo)
            hi = jnp.where(conv & jnp.logical_not(ok), mid - _ONEI32, hi)

        @pl.when(s == 0)
        def _():
            stage[pl.ds(0, 16)] = jnp.full((16,), lo, jnp.int32)
            pltpu.sync_copy(stage.at[pl.ds(0, 16)], out_hbm)

    return sel(x, t)


_ROWS = 512
_COLS = 1024


def _tc_body(x_ref, t_ref, v_ref, out_ref):
    x = x_ref[...]
    t = t_ref[...]
    v = v_ref[0]
    b = jax.lax.bitcast_convert_type(x, jnp.int32)
    key = jnp.where(b >= 0, b, b ^ _MAXI32)
    is_pos = t > 0.0
    key = jnp.where(is_pos, _MINI32, key)
    pos_num_f = jnp.sum(jnp.where(is_pos, 1.0, 0.0))
    k = (pos_num_f * 3.0).astype(jnp.int32)

    vb = jnp.where(v >= 0, v, v ^ _MAXI32)
    x_v = jax.lax.bitcast_convert_type(vb, jnp.float32)

    softplus = jnp.maximum(x, 0.0) + jnp.log1p(jnp.exp(-jnp.abs(x)))
    gt = key > v
    count_gt = jnp.sum(jnp.where(gt, 1, 0))
    sum_gt = jnp.sum(jnp.where(gt, softplus, 0.0))
    pos_sum = jnp.sum(jnp.where(is_pos, softplus - x * t, 0.0))
    sp_v = jnp.maximum(x_v, 0.0) + jnp.log1p(jnp.exp(-jnp.abs(x_v)))
    tie_sum = (k - count_gt).astype(jnp.float32) * sp_v
    total = pos_num_f + k.astype(jnp.float32)
    out_ref[0, 0] = (pos_sum + sum_gt + tie_sum) / total


def kernel(input, target):
    v = _sc_select(input, target)
    x2 = input.reshape(_ROWS, _COLS)
    t2 = target.reshape(_ROWS, _COLS)
    out = pl.pallas_call(
        _tc_body,
        out_shape=jax.ShapeDtypeStruct((1, 1), jnp.float32),
        in_specs=[
            pl.BlockSpec(memory_space=pltpu.VMEM),
            pl.BlockSpec(memory_space=pltpu.VMEM),
            pl.BlockSpec(memory_space=pltpu.SMEM),
        ],
        out_specs=pl.BlockSpec(memory_space=pltpu.SMEM),
    )(x2, t2, v)
    return out[0, 0]


# speedup vs baseline: 1.1406x; 1.0315x over previous
import functools

import jax
import jax.numpy as jnp
import numpy as np
from jax import lax
from jax.experimental import pallas as pl
from jax.experimental.pallas import tpu as pltpu
from jax.experimental.pallas import tpu_sc as plsc

_N = 524288
_NW = 16
_CHUNK = _N // _NW
_VPC = _CHUNK // 16
_UNROLL = 8
_ROUNDS = 32
_MINI32 = np.int32(-2147483648)
_MAXI32 = np.int32(0x7FFFFFFF)
_ONEI32 = np.int32(1)


def _fkey(f):
    return np.frombuffer(np.float32(f).tobytes(), np.int32)[0]


_KA = _fkey(0.90)
_KB = _fkey(1.03)
_KC = _fkey(1.12)


def _keyify(x, t):
    b = lax.bitcast_convert_type(x, jnp.int32)
    key = jnp.where(b >= 0, b, b ^ _MAXI32)
    return jnp.where(t > 0.0, _MINI32, key)


def _sc_select(x, t):
    mesh = plsc.VectorSubcoreMesh(
        core_axis_name="c", subcore_axis_name="s", num_cores=1
    )

    @functools.partial(
        pl.kernel,
        out_type=jax.ShapeDtypeStruct((16,), jnp.int32),
        mesh=mesh,
        compiler_params=pltpu.CompilerParams(needs_layout_passes=False),
        scratch_types=[
            pltpu.VMEM((_CHUNK,), jnp.float32),
            pltpu.VMEM((_CHUNK,), jnp.float32),
            pltpu.VMEM((_CHUNK,), jnp.int32),
            pltpu.VMEM((64,), jnp.int32),
            pltpu.VMEM((1024,), jnp.int32),
            pltpu.VMEM_SHARED(((_ROUNDS + 2) * 1024,), jnp.int32),
        ],
    )
    def sel(x_hbm, t_hbm, out_hbm, xv, tv, kv, stage, rd, shared):
        s = lax.axis_index("s")
        base = s * _CHUNK
        pltpu.sync_copy(x_hbm.at[pl.ds(base, _CHUNK)], xv)
        pltpu.sync_copy(t_hbm.at[pl.ds(base, _CHUNK)], tv)

        def merge(slot, rows):
            for ri, row in enumerate(rows):
                stage[pl.ds(ri * 16, 16)] = row
            pltpu.sync_copy(stage, shared.at[pl.ds(slot * 1024 + s * 64, 64)])
            plsc.subcore_barrier()
            pltpu.sync_copy(shared.at[pl.ds(slot * 1024, 1024)], rd)
            out = []
            for ri in range(len(rows)):
                tot = rd[pl.ds(ri * 16, 16)]
                for j in range(1, 16):
                    tot = tot + rd[pl.ds(j * 64 + ri * 16, 16)]
                out.append(tot)
            return out

        def p0(i, pc):
            o = i * (16 * _UNROLL)
            for j in range(_UNROLL):
                xj = xv[pl.ds(o + j * 16, 16)]
                tj = tv[pl.ds(o + j * 16, 16)]
                kv[pl.ds(o + j * 16, 16)] = _keyify(xj, tj)
                pc = pc + plsc.all_reduce_population_count(tj > 0.0)
            return pc

        pos_part = lax.fori_loop(
            0, _VPC // _UNROLL, p0, jnp.zeros((16,), jnp.int32)
        )

        def bstep(i, accs):
            aa, ab, ac = accs
            o = i * (16 * _UNROLL)
            for j in range(_UNROLL):
                u = kv[pl.ds(o + j * 16, 16)]
                aa = aa + plsc.all_reduce_population_count(u >= _KA)
                ab = ab + plsc.all_reduce_population_count(u >= _KB)
                ac = ac + plsc.all_reduce_population_count(u >= _KC)
            return (aa, ab, ac)

        z = jnp.zeros((16,), jnp.int32)
        ba, bb, bc = lax.fori_loop(0, _VPC // _UNROLL, bstep, (z, z, z))
        g = merge(0, [ba, bb, bc, pos_part])
        pos_num = g[3][0]
        ki = pos_num * np.int32(3)
        ca, cb, cc = g[0][0], g[1][0], g[2][0]

        ok_in = (ca >= ki) & (cc < ki)
        use_b = cb >= ki
        lo = jnp.where(ok_in, jnp.where(use_b, _KB, _KA),
                       jnp.full((), _MINI32 + _ONEI32, jnp.int32))
        hi = jnp.where(ok_in,
                       jnp.where(use_b, _KC - _ONEI32, _KB - _ONEI32),
                       jnp.full((), _MAXI32, jnp.int32))

        for r in range(_ROUNDS):
            d = hi - lo
            mid = lo + lax.shift_right_logical(d, _ONEI32) + (d & _ONEI32)
            conv = hi > lo

            def cpass(mid=mid):
                def cstep(i, accs, mid=mid):
                    a0, a1 = accs
                    o = i * (16 * _UNROLL)
                    for j in range(_UNROLL):
                        u = kv[pl.ds(o + j * 16, 16)]
                        pc = plsc.all_reduce_population_count(u >= mid)
                        if j % 2 == 0:
                            a0 = a0 + pc
                        else:
                            a1 = a1 + pc
                    return (a0, a1)

                a0, a1 = lax.fori_loop(
                    0, _VPC // _UNROLL, cstep,
                    (jnp.zeros((16,), jnp.int32), jnp.zeros((16,), jnp.int32)))
                return a0 + a1

            def full_round(r=r, mid=mid):
                return merge(r + 1, [cpass(mid=mid)])[0][0]

            cnt = lax.cond(conv, full_round, lambda: jnp.zeros((), jnp.int32))
            ok = conv & (cnt >= ki)
            lo = jnp.where(ok, mid, lo)
            hi = jnp.where(conv & jnp.logical_not(ok), mid - _ONEI32, hi)

        @pl.when(s == 0)
        def _():
            stage[pl.ds(0, 16)] = jnp.full((16,), lo, jnp.int32)
            pltpu.sync_copy(stage.at[pl.ds(0, 16)], out_hbm)

    return sel(x, t)


_ROWS = 512
_COLS = 1024


def _tc_body(x_ref, t_ref, v_ref, out_ref):
    x = x_ref[...]
    t = t_ref[...]
    v = v_ref[0]
    b = jax.lax.bitcast_convert_type(x, jnp.int32)
    key = jnp.where(b >= 0, b, b ^ _MAXI32)
    is_pos = t > 0.0
    key = jnp.where(is_pos, _MINI32, key)
    pos_num_f = jnp.sum(jnp.where(is_pos, 1.0, 0.0))
    k = (pos_num_f * 3.0).astype(jnp.int32)

    vb = jnp.where(v >= 0, v, v ^ _MAXI32)
    x_v = jax.lax.bitcast_convert_type(vb, jnp.float32)

    softplus = jnp.maximum(x, 0.0) + jnp.log1p(jnp.exp(-jnp.abs(x)))
    gt = key > v
    count_gt = jnp.sum(jnp.where(gt, 1, 0))
    sum_gt = jnp.sum(jnp.where(gt, softplus, 0.0))
    pos_sum = jnp.sum(jnp.where(is_pos, softplus - x * t, 0.0))
    sp_v = jnp.maximum(x_v, 0.0) + jnp.log1p(jnp.exp(-jnp.abs(x_v)))
    tie_sum = (k - count_gt).astype(jnp.float32) * sp_v
    total = pos_num_f + k.astype(jnp.float32)
    out_ref[0, 0] = (pos_sum + sum_gt + tie_sum) / total


def kernel(input, target):
    v = _sc_select(input, target)
    x2 = input.reshape(_ROWS, _COLS)
    t2 = target.reshape(_ROWS, _COLS)
    out = pl.pallas_call(
        _tc_body,
        out_shape=jax.ShapeDtypeStruct((1, 1), jnp.float32),
        in_specs=[
            pl.BlockSpec(memory_space=pltpu.VMEM),
            pl.BlockSpec(memory_space=pltpu.VMEM),
            pl.BlockSpec(memory_space=pltpu.SMEM),
        ],
        out_specs=pl.BlockSpec(memory_space=pltpu.SMEM),
    )(x2, t2, v)
    return out[0, 0]
